# baseline (speedup 1.0000x reference)
import numpy as np
import jax
import jax.numpy as jnp
from jax.experimental import pallas as pl
from jax.experimental.pallas import tpu as pltpu

_NJ = 14
_COL = 14
_XP = 16
_CP = _COL * _XP
_B = 4096
_BBL = 256


def _blob_table() -> np.ndarray:
    radius = 4
    xs = np.arange(-radius, radius + 1)
    k = np.exp(-0.5 * xs.astype(np.float64) ** 2)
    k = k / k.sum()
    eye = np.eye(_COL)
    eyep = np.pad(eye, ((0, 0), (radius, radius)), mode="symmetric")
    c = np.zeros((_COL, _COL))
    for i in range(2 * radius + 1):
        c = c + k[i] * eyep[:, i : i + _COL]
    full = np.einsum("py,qx->yxpq", c, c)
    t = np.zeros((_COL, _XP, _COL, _XP))
    t[:, :_COL, :, :_COL] = full
    t2 = t.reshape(_CP, _CP)
    aux = np.stack([t2.max(axis=0), (t2 * t2).sum(axis=0)], axis=0)
    return np.concatenate([t2, aux], axis=0).astype(np.float32)


_T2 = _blob_table()


def _mse_kernel(p0_ref, p1_ref, p2_ref, t2_ref, h_ref, out_ref):
    bbl = h_ref.shape[3]
    ii = jax.lax.broadcasted_iota(jnp.int32, (_CP, bbl), 0)
    t2 = t2_ref[...]
    acc = jnp.zeros((1, bbl), jnp.float32)
    cnt = jnp.zeros((1, bbl), jnp.float32)
    for c in range(18):
        p0 = p0_ref[c : c + 1, :]
        if c < _NJ:
            d = ii == p0
        else:
            p1 = p1_ref[c - _NJ : c - _NJ + 1, :]
            p2 = p2_ref[c - _NJ : c - _NJ + 1, :]
            d = (ii == p0) | (ii == p1) | (ii == p2)
        dm = d.astype(jnp.bfloat16)
        f = jnp.dot(t2, dm, preferred_element_type=jnp.float32)
        fm = f[:_CP]
        fv = fm.reshape(_COL, _XP, bbl)[:, :_COL, :]
        hc = h_ref[c]
        if c < _NJ:
            m = f[_CP : _CP + 1]
            s2 = f[_CP + 1 : _CP + 2]
        else:
            m = jnp.max(fm, axis=0, keepdims=True)
            s2 = jnp.sum(fm * fm, axis=0, keepdims=True)
        den = jnp.where(m > 0.0, m, 1.0)
        s1 = jnp.sum(hc * fv, axis=(0, 1))[None, :]
        sh2 = jnp.sum(hc * hc, axis=(0, 1))[None, :]
        mask = (m > 0.0).astype(jnp.float32)
        acc = acc + mask * (sh2 - 2.0 * (s1 / den) + s2 / (den * den))
        cnt = cnt + mask
    out_ref[0, 0, 0] = jnp.sum(acc)
    out_ref[0, 0, 1] = jnp.sum(cnt)


def kernel(os_, h, t, v):
    del os_
    b = h.shape[0]
    grid = b // _BBL
    ht = jnp.transpose(h, (1, 2, 3, 0))
    ti = t * float(_COL)
    xi = jnp.clip(ti[:, :, 0].astype(jnp.int32), 0, _COL - 1)
    yi = jnp.clip(ti[:, :, 1].astype(jnp.int32), 0, _COL - 1)
    vis = v[:, :, 0] == 1.0
    posv = jnp.where(vis, yi * _XP + xi, -1)
    pj = posv.T
    p0 = jnp.concatenate([pj, pj[0:12:3]], axis=0)
    p1 = pj[1:12:3]
    p2 = pj[2:12:3]
    t2 = jnp.asarray(_T2, dtype=jnp.bfloat16)
    partial = pl.pallas_call(
        _mse_kernel,
        grid=(grid,),
        in_specs=[
            pl.BlockSpec((18, _BBL), lambda i: (0, i)),
            pl.BlockSpec((4, _BBL), lambda i: (0, i)),
            pl.BlockSpec((4, _BBL), lambda i: (0, i)),
            pl.BlockSpec((_CP + 2, _CP), lambda i: (0, 0)),
            pl.BlockSpec((18, _COL, _COL, _BBL), lambda i: (0, 0, 0, i)),
        ],
        out_specs=pl.BlockSpec(
            (1, 1, 2), lambda i: (i, 0, 0), memory_space=pltpu.SMEM
        ),
        out_shape=jax.ShapeDtypeStruct((grid, 1, 2), jnp.float32),
        compiler_params=pltpu.CompilerParams(
            dimension_semantics=("parallel",),
        ),
    )(p0, p1, p2, t2, ht)
    total = jnp.sum(partial[:, 0, 0])
    cnt = jnp.sum(partial[:, 0, 1])
    return total / (cnt * float(_COL * _COL))


# speedup vs baseline: 1.2649x; 1.2649x over previous
import numpy as np
import jax
import jax.numpy as jnp
from jax.experimental import pallas as pl
from jax.experimental.pallas import tpu as pltpu

_NJ = 14
_COL = 14
_XP = 16
_CP = _COL * _XP
_B = 4096
_BBL = 512


def _blob_table() -> np.ndarray:
    radius = 4
    xs = np.arange(-radius, radius + 1)
    k = np.exp(-0.5 * xs.astype(np.float64) ** 2)
    k = k / k.sum()
    eye = np.eye(_COL)
    eyep = np.pad(eye, ((0, 0), (radius, radius)), mode="symmetric")
    c = np.zeros((_COL, _COL))
    for i in range(2 * radius + 1):
        c = c + k[i] * eyep[:, i : i + _COL]
    full = np.einsum("py,qx->yxpq", c, c)
    t = np.zeros((_COL, _XP, _COL, _XP))
    t[:, :_COL, :, :_COL] = full
    t2 = t.reshape(_CP, _CP)
    aux = np.stack([t2.max(axis=0), (t2 * t2).sum(axis=0)], axis=0)
    return np.concatenate([t2, aux], axis=0).astype(np.float32)


_T2 = _blob_table()


def _mse_kernel(t_ref, v_ref, t2_ref, h_ref, out_ref):
    bbl = h_ref.shape[3]
    ii = jax.lax.broadcasted_iota(jnp.int32, (_CP, bbl), 0)
    t2 = t2_ref[...]
    ti = t_ref[...] * float(_COL)
    xi = jnp.clip(ti[:, 0, :].astype(jnp.int32), 0, _COL - 1)
    yi = jnp.clip(ti[:, 1, :].astype(jnp.int32), 0, _COL - 1)
    vis = v_ref[:, 0, :] == 1.0
    posv = jnp.where(vis, yi * _XP + xi, -1)
    acc = jnp.zeros((1, bbl), jnp.float32)
    cnt = jnp.zeros((1, bbl), jnp.float32)
    for c in range(18):
        if c < _NJ:
            d = ii == posv[c : c + 1, :]
        else:
            g = 3 * (c - _NJ)
            d = (
                (ii == posv[g : g + 1, :])
                | (ii == posv[g + 1 : g + 2, :])
                | (ii == posv[g + 2 : g + 3, :])
            )
        dm = d.astype(jnp.bfloat16)
        f = jnp.dot(t2, dm, preferred_element_type=jnp.float32)
        fm = f[:_CP]
        fv = fm.reshape(_COL, _XP, bbl)[:, :_COL, :]
        hc = h_ref[c]
        if c < _NJ:
            m = f[_CP : _CP + 1]
            s2 = f[_CP + 1 : _CP + 2]
        else:
            m = jnp.max(fm, axis=0, keepdims=True)
            s2 = jnp.sum(fm * fm, axis=0, keepdims=True)
        den = jnp.where(m > 0.0, m, 1.0)
        s1 = jnp.sum(hc * fv, axis=(0, 1))[None, :]
        sh2 = jnp.sum(hc * hc, axis=(0, 1))[None, :]
        mask = (m > 0.0).astype(jnp.float32)
        acc = acc + mask * (sh2 - 2.0 * (s1 / den) + s2 / (den * den))
        cnt = cnt + mask
    out_ref[0, 0, 0] = jnp.sum(acc)
    out_ref[0, 0, 1] = jnp.sum(cnt)


def kernel(os_, h, t, v):
    del os_
    b = h.shape[0]
    grid = b // _BBL
    ht = jnp.transpose(h, (1, 2, 3, 0))
    tt = jnp.transpose(t, (1, 2, 0))
    vt = jnp.transpose(v, (1, 2, 0))
    t2 = jnp.asarray(_T2, dtype=jnp.bfloat16)
    partial = pl.pallas_call(
        _mse_kernel,
        grid=(grid,),
        in_specs=[
            pl.BlockSpec((_NJ, 2, _BBL), lambda i: (0, 0, i)),
            pl.BlockSpec((_NJ, 2, _BBL), lambda i: (0, 0, i)),
            pl.BlockSpec((_CP + 2, _CP), lambda i: (0, 0)),
            pl.BlockSpec((18, _COL, _COL, _BBL), lambda i: (0, 0, 0, i)),
        ],
        out_specs=pl.BlockSpec(
            (1, 1, 2), lambda i: (i, 0, 0), memory_space=pltpu.SMEM
        ),
        out_shape=jax.ShapeDtypeStruct((grid, 1, 2), jnp.float32),
        compiler_params=pltpu.CompilerParams(
            dimension_semantics=("parallel",),
        ),
    )(tt, vt, t2, ht)
    total = jnp.sum(partial[:, 0, 0])
    cnt = jnp.sum(partial[:, 0, 1])
    return total / (cnt * float(_COL * _COL))
